# Initial kernel scaffold; baseline (speedup 1.0000x reference)
import jax
import jax.numpy as jnp
from jax.experimental import pallas as pl


def kernel(x, edge_index, edge_weight1, edge_weight2, W1, b1, W2, b2):
    raise NotImplementedError("write your pallas kernel here")


# speedup vs baseline: 4.1087x; 4.1087x over previous
import functools

import jax
import jax.numpy as jnp
from jax import lax
from jax.experimental import pallas as pl
from jax.experimental.pallas import tpu as pltpu
from jax.experimental.pallas import tpu_sc as plsc

_NC = 2
_NS = 16
_NW = _NC * _NS
_CH = 128
_LANES = 16


def _sc_mesh():
    return plsc.VectorSubcoreMesh(core_axis_name="c", subcore_axis_name="s")


@functools.lru_cache(maxsize=None)
def _make_deg_kernel(C, n_pad):
    R = n_pad // _NS

    @functools.partial(
        pl.kernel,
        mesh=_sc_mesh(),
        out_type=[jax.ShapeDtypeStruct((_NC, n_pad), jnp.float32),
                  jax.ShapeDtypeStruct((_NC, n_pad), jnp.float32)],
        scratch_types=[
            pltpu.VMEM((C, _CH), jnp.int32),
            pltpu.VMEM((C, _CH), jnp.int32),
            pltpu.VMEM((_CH,), jnp.float32),
            pltpu.VMEM((R,), jnp.float32),
            pltpu.VMEM_SHARED((n_pad,), jnp.float32),
            pltpu.VMEM_SHARED((n_pad,), jnp.float32),
        ],
    )
    def deg_kernel(src_hbm, dst_hbm, od_hbm, id_hbm,
                   src_v, dst_v, ones_v, zb_v, sh_od, sh_id):
        cid = lax.axis_index("c")
        sid = lax.axis_index("s")
        w = sid * _NC + cid
        pltpu.sync_copy(src_hbm.at[w], src_v)
        pltpu.sync_copy(dst_hbm.at[w], dst_v)
        for g in range(_CH // _LANES):
            ones_v[pl.ds(g * _LANES, _LANES)] = jnp.ones((_LANES,), jnp.float32)

        def zb(i, carry):
            zb_v[pl.ds(i * _LANES, _LANES)] = jnp.zeros((_LANES,), jnp.float32)
            return carry
        lax.fori_loop(0, R // _LANES, zb, 0)

        base = sid * R
        pltpu.sync_copy(zb_v, sh_od.at[pl.ds(base, R)])
        pltpu.sync_copy(zb_v, sh_id.at[pl.ds(base, R)])
        plsc.subcore_barrier()

        def body(j, carry):
            pltpu.sync_copy(ones_v, sh_od.at[src_v.at[j]], add=True)
            pltpu.sync_copy(ones_v, sh_id.at[dst_v.at[j]], add=True)
            return carry
        lax.fori_loop(0, C, body, 0)
        plsc.subcore_barrier()

        pltpu.sync_copy(sh_od.at[pl.ds(base, R)], od_hbm.at[cid, pl.ds(base, R)])
        pltpu.sync_copy(sh_id.at[pl.ds(base, R)], id_hbm.at[cid, pl.ds(base, R)])

    return deg_kernel


@functools.lru_cache(maxsize=None)
def _make_spmm_kernel(C, n_pad, D):
    R = n_pad // _NS
    RB = R // _CH

    @functools.partial(
        pl.kernel,
        mesh=_sc_mesh(),
        out_type=jax.ShapeDtypeStruct((_NC, n_pad, D), jnp.float32),
        scratch_types=[
            pltpu.VMEM((C, _CH), jnp.int32),
            pltpu.VMEM((C, _CH), jnp.int32),
            pltpu.VMEM((C, _CH), jnp.float32),
            pltpu.VMEM((_CH, D), jnp.float32),
            pltpu.VMEM((_CH,), jnp.float32),
            pltpu.VMEM((_CH,), jnp.float32),
            pltpu.VMEM_SHARED((n_pad, D), jnp.float32),
            pltpu.SemaphoreType.DMA,
        ],
    )
    def spmm_kernel(feat_hbm, src_hbm, dst_hbm, ew_hbm, rso_hbm, rsi_hbm,
                    out_hbm, src_v, dst_v, cf_v, rows_v, ga_v, gb_v,
                    acc, sem):
        cid = lax.axis_index("c")
        sid = lax.axis_index("s")
        w = sid * _NC + cid
        pltpu.sync_copy(src_hbm.at[w], src_v)
        pltpu.sync_copy(dst_hbm.at[w], dst_v)
        pltpu.sync_copy(ew_hbm.at[w], cf_v)

        def zr(i, carry):
            for f in range(D // _LANES):
                rows_v[i, pl.ds(f * _LANES, _LANES)] = (
                    jnp.zeros((_LANES,), jnp.float32))
            return carry
        lax.fori_loop(0, _CH, zr, 0)

        base = sid * R
        for kb in range(RB):
            pltpu.sync_copy(rows_v, acc.at[pl.ds(base + kb * _CH, _CH)])

        def cf(j, carry):
            pltpu.async_copy(rso_hbm.at[src_v.at[j]], ga_v, sem).wait()
            pltpu.async_copy(rsi_hbm.at[dst_v.at[j]], gb_v, sem).wait()
            for g in range(_CH // _LANES):
                sl = pl.ds(g * _LANES, _LANES)
                cf_v[j, sl] = cf_v[j, sl] * ga_v[sl] * gb_v[sl]
            return carry
        lax.fori_loop(0, C, cf, 0)
        plsc.subcore_barrier()

        def chunk(j, carry):
            pltpu.async_copy(feat_hbm.at[src_v.at[j]], rows_v, sem).wait()

            def grp(g, c2):
                c16 = cf_v[j, pl.ds(g * _LANES, _LANES)]
                for e2 in range(_LANES):
                    ce = c16[e2]
                    e = g * _LANES + e2
                    for f in range(D // _LANES):
                        sl = pl.ds(f * _LANES, _LANES)
                        rows_v[e, sl] = rows_v[e, sl] * ce
                return c2
            lax.fori_loop(0, _CH // _LANES, grp, 0)
            pltpu.sync_copy(rows_v, acc.at[dst_v.at[j]], add=True)
            return carry
        lax.fori_loop(0, C, chunk, 0)
        plsc.subcore_barrier()

        for kb in range(RB):
            sl = pl.ds(base + kb * _CH, _CH)
            pltpu.sync_copy(acc.at[sl], out_hbm.at[cid, sl])

    return spmm_kernel


def _mm_fused(p0, p1, W1, b1, W2):
    n_pad, d_in = p0.shape
    d_h = W1.shape[1]
    d_out = W2.shape[1]
    blk = 1024

    def body(p0_r, p1_r, w1_r, b1_r, w2_r, o_r):
        h = jnp.dot(p0_r[...] + p1_r[...], w1_r[...],
                    preferred_element_type=jnp.float32)
        h = jnp.maximum(h + b1_r[...], 0.0)
        o_r[...] = jnp.dot(h, w2_r[...], preferred_element_type=jnp.float32)

    return pl.pallas_call(
        body,
        grid=(n_pad // blk,),
        in_specs=[
            pl.BlockSpec((blk, d_in), lambda i: (i, 0)),
            pl.BlockSpec((blk, d_in), lambda i: (i, 0)),
            pl.BlockSpec((d_in, d_h), lambda i: (0, 0)),
            pl.BlockSpec((1, d_h), lambda i: (0, 0)),
            pl.BlockSpec((d_h, d_out), lambda i: (0, 0)),
        ],
        out_specs=pl.BlockSpec((blk, d_out), lambda i: (i, 0)),
        out_shape=jax.ShapeDtypeStruct((n_pad, d_out), jnp.float32),
    )(p0, p1, W1, b1.reshape(1, -1), W2)


def _bias_relu(p0, p1, b):
    n_pad, d = p0.shape
    blk = 1024

    def body(p0_r, p1_r, b_r, o_r):
        o_r[...] = jnp.maximum(p0_r[...] + p1_r[...] + b_r[...], 0.0)

    return pl.pallas_call(
        body,
        grid=(n_pad // blk,),
        in_specs=[
            pl.BlockSpec((blk, d), lambda i: (i, 0)),
            pl.BlockSpec((blk, d), lambda i: (i, 0)),
            pl.BlockSpec((1, d), lambda i: (0, 0)),
        ],
        out_specs=pl.BlockSpec((blk, d), lambda i: (i, 0)),
        out_shape=jax.ShapeDtypeStruct((n_pad, d), jnp.float32),
    )(p0, p1, b.reshape(1, -1))


def kernel(x, edge_index, edge_weight1, edge_weight2, W1, b1, W2, b2):
    n, d_in = x.shape
    e = edge_index.shape[1]
    d_out = W2.shape[1]

    rows_per_tile = _NS * _CH
    n_pad = ((n + rows_per_tile - 1) // rows_per_tile) * rows_per_tile
    C = (e + _NW * _CH - 1) // (_NW * _CH)
    e_pad = C * _NW * _CH

    pad = e_pad - e
    src_p = jnp.concatenate(
        [edge_index[0], jnp.full((pad,), n, jnp.int32)]).reshape(_NW, C, _CH)
    dst_p = jnp.concatenate(
        [edge_index[1], jnp.full((pad,), n, jnp.int32)]).reshape(_NW, C, _CH)
    ew1_p = jnp.concatenate(
        [edge_weight1, jnp.zeros((pad,), jnp.float32)]).reshape(_NW, C, _CH)
    ew2_p = jnp.concatenate(
        [edge_weight2, jnp.zeros((pad,), jnp.float32)]).reshape(_NW, C, _CH)
    x_p = jnp.pad(x, ((0, n_pad - n), (0, 0)))

    od, idg = _make_deg_kernel(C, n_pad)(src_p, dst_p)
    rs_out = lax.rsqrt(jnp.maximum(od[0] + od[1], 1.0))
    rs_in = lax.rsqrt(jnp.maximum(idg[0] + idg[1], 1.0))

    spmm = _make_spmm_kernel(C, n_pad, d_in)
    h1 = spmm(x_p, src_p, dst_p, ew1_p, rs_out, rs_in)
    y2 = _mm_fused(h1[0], h1[1], W1, b1, W2)
    h2 = _make_spmm_kernel(C, n_pad, d_out)(
        y2, src_p, dst_p, ew2_p, rs_out, rs_in)
    out = _bias_relu(h2[0], h2[1], b2)
    return out[:n]
